# baseline (speedup 1.0000x reference)
import jax
import jax.numpy as jnp
from jax import lax
from jax.experimental import pallas as pl
from jax.experimental.pallas import tpu as pltpu
from jax.experimental.pallas import tpu_sc as plsc

_NC = 2
_NS = 16
_NW = _NC * _NS
_LANES = 16
_CHUNK = 32


def _emb_body(ids_hbm, tok_hbm, pos_hbm, out_hbm,
              idx_v, pos_v, buf0, buf1, buf2,
              sin0, sin1, sin2, sout0, sout1, sout2, spos):
    n = ids_hbm.shape[0]
    hidden = tok_hbm.shape[1]
    max_pos = pos_hbm.shape[0]
    per_w = n // _NW
    nchunk = per_w // _CHUNK
    bufs = (buf0, buf1, buf2)
    sins = (sin0, sin1, sin2)
    souts = (sout0, sout1, sout2)

    wid = lax.axis_index("s") * _NC + lax.axis_index("c")
    base = wid * per_w
    pltpu.sync_copy(ids_hbm.at[pl.ds(base, per_w)], idx_v)
    pos_cp = pltpu.async_copy(pos_hbm, pos_v, spos)

    def gather(x, p):
        pltpu.async_copy(
            tok_hbm.at[idx_v.at[pl.ds(x * _CHUNK, _CHUNK)]], bufs[p], sins[p])

    def wait_in(p):
        pltpu.make_async_copy(
            tok_hbm.at[pl.ds(0, _CHUNK)], bufs[p], sins[p]).wait()

    def put(x, p):
        pltpu.async_copy(
            bufs[p], out_hbm.at[pl.ds(base + x * _CHUNK, _CHUNK)], souts[p])

    def wait_out(p):
        pltpu.make_async_copy(
            bufs[p], out_hbm.at[pl.ds(0, _CHUNK)], souts[p]).wait()

    def vadd(x, p):
        off = lax.rem(x * _CHUNK, max_pos)

        @plsc.parallel_loop(0, _CHUNK, unroll=4)
        def _row(r):
            for j in range(hidden // _LANES):
                sl = pl.ds(j * _LANES, _LANES)
                plsc.addupdate(bufs[p].at[r, sl], pos_v[off + r, sl])

    gather(0, 0)
    gather(1, 1)
    pos_cp.wait()

    @pl.loop(0, nchunk - 3, step=3)
    def _main(c):
        for k in range(3):
            x = c + k
            p = k
            q = (k + 2) % 3
            wait_in(p)
            if k == 0:
                @pl.when(c >= 1)
                def _():
                    wait_out(q)
            else:
                wait_out(q)
            if k == 2:
                @pl.when(c <= nchunk - 5)
                def _():
                    gather(x + 2, q)
            else:
                gather(x + 2, q)
            vadd(x, p)
            put(x, p)

    last = nchunk - 1
    wait_in(0)
    vadd(last, 0)
    wait_out(2)
    put(last, 0)
    wait_out(0)


def kernel(input_ids, token_table, pos_table):
    b, s = input_ids.shape
    hidden = token_table.shape[1]
    max_pos = pos_table.shape[0]
    n = b * s
    ids_flat = input_ids.reshape(n).astype(jnp.int32)
    per_w = n // _NW

    mesh = plsc.VectorSubcoreMesh(core_axis_name="c", subcore_axis_name="s")
    run = pl.kernel(
        _emb_body,
        out_type=jax.ShapeDtypeStruct((n, hidden), jnp.float32),
        mesh=mesh,
        scratch_types=[
            pltpu.VMEM((per_w,), jnp.int32),
            pltpu.VMEM((max_pos, hidden), jnp.float32),
            pltpu.VMEM((_CHUNK, hidden), jnp.float32),
            pltpu.VMEM((_CHUNK, hidden), jnp.float32),
            pltpu.VMEM((_CHUNK, hidden), jnp.float32),
            pltpu.SemaphoreType.DMA,
            pltpu.SemaphoreType.DMA,
            pltpu.SemaphoreType.DMA,
            pltpu.SemaphoreType.DMA,
            pltpu.SemaphoreType.DMA,
            pltpu.SemaphoreType.DMA,
            pltpu.SemaphoreType.DMA,
        ],
    )
    out = run(ids_flat, token_table, pos_table)
    return out.reshape(b, s, hidden)


# speedup vs baseline: 1.6310x; 1.6310x over previous
import jax
import jax.numpy as jnp
from jax import lax
from jax.experimental import pallas as pl
from jax.experimental.pallas import tpu as pltpu
from jax.experimental.pallas import tpu_sc as plsc

_NC = 2
_NS = 16
_NW = _NC * _NS
_LANES = 16
_CHUNK = 32
_NBUF = 4


def _emb_body(ids_hbm, tok_hbm, pos_hbm, out_hbm,
              idx_v, pos_v, buf0, buf1, buf2, buf3,
              sin0, sin1, sin2, sin3, sout0, sout1, sout2, sout3, spos, sidx):
    n = ids_hbm.shape[0]
    hidden = tok_hbm.shape[1]
    seq = pos_hbm.shape[0]
    batch = n // seq
    bufs = (buf0, buf1, buf2, buf3)
    sins = (sin0, sin1, sin2, sin3)
    souts = (sout0, sout1, sout2, sout3)

    wid = lax.axis_index("s") * _NC + lax.axis_index("c")
    half = lax.rem(wid, 2)
    bgroup = wid // 2
    nbatch = batch // (_NW // 2)
    nchunk = nbatch
    b0 = bgroup * nbatch
    poff = half * _CHUNK

    for j in range(nchunk):
        pltpu.async_copy(
            ids_hbm.at[pl.ds((b0 + j) * seq + poff, _CHUNK)],
            idx_v.at[pl.ds(j * _CHUNK, _CHUNK)], sidx)
    pos_cp = pltpu.async_copy(
        pos_hbm.at[pl.ds(poff, _CHUNK)], pos_v, spos)
    pltpu.make_async_copy(
        ids_hbm.at[pl.ds(0, nchunk * _CHUNK)], idx_v, sidx).wait()

    def gather(x, p):
        pltpu.async_copy(
            tok_hbm.at[idx_v.at[pl.ds(x * _CHUNK, _CHUNK)]],
            bufs[p], sins[p])

    def wait_in(p):
        pltpu.make_async_copy(
            tok_hbm.at[pl.ds(0, _CHUNK)], bufs[p], sins[p]).wait()

    def put(x, p):
        pltpu.async_copy(
            bufs[p], out_hbm.at[pl.ds((b0 + x) * seq + poff, _CHUNK)],
            souts[p])

    def wait_out(p):
        pltpu.make_async_copy(
            bufs[p], out_hbm.at[pl.ds(0, _CHUNK)], souts[p]).wait()

    def vadd(p):
        @plsc.parallel_loop(0, _CHUNK, unroll=2)
        def _row(r):
            for j in range(hidden // _LANES):
                sl = pl.ds(j * _LANES, _LANES)
                plsc.addupdate(bufs[p].at[r, sl], pos_v[r, sl])

    gather(0, 0)
    gather(1, 1)
    gather(2, 2)
    pos_cp.wait()

    @pl.loop(0, nchunk, step=_NBUF)
    def _main(c):
        for k in range(_NBUF):
            x = c + k
            p = k
            q = (k + 3) % _NBUF
            wait_in(p)
            vadd(p)
            put(x, p)
            if k == 0:
                @pl.when(c >= 1)
                def _():
                    wait_out(q)
            else:
                wait_out(q)
            if k == 0:
                gather(x + 3, q)
            else:
                @pl.when(x + 3 <= nchunk - 1)
                def _():
                    gather(x + 3, q)

    wait_out(3)


def kernel(input_ids, token_table, pos_table):
    b, s = input_ids.shape
    hidden = token_table.shape[1]
    n = b * s
    ids = input_ids.reshape(n).astype(jnp.int32)
    nbatch = b // (_NW // 2)

    mesh = plsc.VectorSubcoreMesh(core_axis_name="c", subcore_axis_name="s")
    run = pl.kernel(
        _emb_body,
        out_type=jax.ShapeDtypeStruct((n, hidden), jnp.float32),
        mesh=mesh,
        scratch_types=[
            pltpu.VMEM((nbatch * _CHUNK,), jnp.int32),
            pltpu.VMEM((_CHUNK, hidden), jnp.float32),
            pltpu.VMEM((_CHUNK, hidden), jnp.float32),
            pltpu.VMEM((_CHUNK, hidden), jnp.float32),
            pltpu.VMEM((_CHUNK, hidden), jnp.float32),
            pltpu.VMEM((_CHUNK, hidden), jnp.float32),
            pltpu.SemaphoreType.DMA,
            pltpu.SemaphoreType.DMA,
            pltpu.SemaphoreType.DMA,
            pltpu.SemaphoreType.DMA,
            pltpu.SemaphoreType.DMA,
            pltpu.SemaphoreType.DMA,
            pltpu.SemaphoreType.DMA,
            pltpu.SemaphoreType.DMA,
            pltpu.SemaphoreType.DMA,
            pltpu.SemaphoreType.DMA,
        ],
    )
    out = run(ids, token_table, pos_table)
    return out.reshape(b, s, hidden)


# speedup vs baseline: 1.6468x; 1.0097x over previous
import jax
import jax.numpy as jnp
from jax import lax
from jax.experimental import pallas as pl
from jax.experimental.pallas import tpu as pltpu
from jax.experimental.pallas import tpu_sc as plsc

_NC = 2
_NS = 16
_NW = _NC * _NS
_LANES = 16
_CHUNK = 32
_NBUF = 4


def _emb_body(ids_hbm, tok_hbm, pos_hbm, out_hbm,
              idx_v, pos_v, buf0, buf1, buf2, buf3,
              sin0, sin1, sin2, sin3, sout0, sout1, sout2, sout3, spos):
    n = ids_hbm.shape[0]
    hidden = tok_hbm.shape[1]
    seq = pos_hbm.shape[0]
    batch = n // seq
    bufs = (buf0, buf1, buf2, buf3)
    sins = (sin0, sin1, sin2, sin3)
    souts = (sout0, sout1, sout2, sout3)

    wid = lax.axis_index("s") * _NC + lax.axis_index("c")
    half = lax.rem(wid, 2)
    bgroup = wid // 2
    nbatch = batch // (_NW // 2)
    nchunk = nbatch
    b0 = bgroup * nbatch
    poff = half * _CHUNK

    pos_cp = pltpu.async_copy(
        pos_hbm.at[pl.ds(poff, _CHUNK)], pos_v, spos)
    pltpu.sync_copy(
        ids_hbm.at[pl.ds(wid * nchunk * _CHUNK, nchunk * _CHUNK)], idx_v)

    def gather(x, p):
        pltpu.async_copy(
            tok_hbm.at[idx_v.at[pl.ds(x * _CHUNK, _CHUNK)]],
            bufs[p], sins[p])

    def wait_in(p):
        pltpu.make_async_copy(
            tok_hbm.at[pl.ds(0, _CHUNK)], bufs[p], sins[p]).wait()

    def put(x, p):
        pltpu.async_copy(
            bufs[p], out_hbm.at[pl.ds((b0 + x) * seq + poff, _CHUNK)],
            souts[p])

    def wait_out(p):
        pltpu.make_async_copy(
            bufs[p], out_hbm.at[pl.ds(0, _CHUNK)], souts[p]).wait()

    def vadd(p):
        @plsc.parallel_loop(0, _CHUNK, unroll=2)
        def _row(r):
            for j in range(hidden // _LANES):
                sl = pl.ds(j * _LANES, _LANES)
                plsc.addupdate(bufs[p].at[r, sl], pos_v[r, sl])

    gather(0, 0)
    gather(1, 1)
    gather(2, 2)
    pos_cp.wait()

    @pl.loop(0, nchunk, step=_NBUF)
    def _main(c):
        for k in range(_NBUF):
            x = c + k
            p = k
            q = (k + 3) % _NBUF
            wait_in(p)
            vadd(p)
            put(x, p)
            if k == 0:
                @pl.when(c >= 1)
                def _():
                    wait_out(q)
            else:
                wait_out(q)
            if k == 0:
                gather(x + 3, q)
            else:
                @pl.when(x + 3 <= nchunk - 1)
                def _():
                    gather(x + 3, q)

    wait_out(3)


def kernel(input_ids, token_table, pos_table):
    b, s = input_ids.shape
    hidden = token_table.shape[1]
    n = b * s
    nbatch = b // (_NW // 2)
    ids = (input_ids.astype(jnp.int32)
           .reshape(_NW // 2, nbatch, 2, _CHUNK)
           .transpose(0, 2, 1, 3)
           .reshape(n))

    mesh = plsc.VectorSubcoreMesh(core_axis_name="c", subcore_axis_name="s")
    run = pl.kernel(
        _emb_body,
        out_type=jax.ShapeDtypeStruct((n, hidden), jnp.float32),
        mesh=mesh,
        scratch_types=[
            pltpu.VMEM((nbatch * _CHUNK,), jnp.int32),
            pltpu.VMEM((_CHUNK, hidden), jnp.float32),
            pltpu.VMEM((_CHUNK, hidden), jnp.float32),
            pltpu.VMEM((_CHUNK, hidden), jnp.float32),
            pltpu.VMEM((_CHUNK, hidden), jnp.float32),
            pltpu.VMEM((_CHUNK, hidden), jnp.float32),
            pltpu.SemaphoreType.DMA,
            pltpu.SemaphoreType.DMA,
            pltpu.SemaphoreType.DMA,
            pltpu.SemaphoreType.DMA,
            pltpu.SemaphoreType.DMA,
            pltpu.SemaphoreType.DMA,
            pltpu.SemaphoreType.DMA,
            pltpu.SemaphoreType.DMA,
            pltpu.SemaphoreType.DMA,
        ],
    )
    out = run(ids, token_table, pos_table)
    return out.reshape(b, s, hidden)
